# baseline (speedup 1.0000x reference)
import functools

import jax
import jax.numpy as jnp
from jax import lax
from jax.experimental import pallas as pl
from jax.experimental.pallas import tpu as pltpu
from jax.experimental.pallas import tpu_sc as plsc

_D = 64
_NW = 32
_SPC = 4
_RING = 8


@functools.lru_cache(maxsize=None)
def _make_gather(batch: int, seq: int):
    toks_per_chunk = _SPC * seq
    assert batch % (_NW * _SPC * _RING) == 0
    chunks_per_w = batch // (_NW * _SPC)
    sents_per_w = batch // _NW
    n_rounds = chunks_per_w // _RING
    mesh = plsc.VectorSubcoreMesh(core_axis_name="c", subcore_axis_name="s")

    scratch = (
        [pltpu.VMEM((sents_per_w * seq,), jnp.int32)]
        + [pltpu.VMEM((toks_per_chunk, _D), jnp.float32) for _ in range(_RING)]
        + [pltpu.SemaphoreType.DMA for _ in range(2 * _RING)]
    )

    @functools.partial(
        pl.kernel,
        mesh=mesh,
        compiler_params=pltpu.CompilerParams(use_tc_tiling_on_sc=False),
        out_type=jax.ShapeDtypeStruct((batch, seq, _D), jnp.float32),
        scratch_types=scratch,
    )
    def gather_kernel(idx_hbm, table_hbm, out_hbm, idx_v, *rest):
        bufs = rest[:_RING]
        gsem = rest[_RING:2 * _RING]
        wsem = rest[2 * _RING:]
        wid = lax.axis_index("s") * 2 + lax.axis_index("c")
        tok0 = wid * (sents_per_w * seq)
        sent0 = wid * sents_per_w
        pltpu.sync_copy(idx_hbm.at[pl.ds(tok0, sents_per_w * seq)], idx_v)

        def round_body(p, carry):
            c = _RING * p
            for j in range(_RING):
                @pl.when(p > 0)
                def _(j=j, c=c):
                    b = sent0 + (c - _RING + j) * _SPC
                    for i in range(_SPC):
                        pltpu.make_async_copy(
                            bufs[j].at[pl.ds(i * seq, seq)],
                            out_hbm.at[b + i], wsem[j]).wait()
                ids = idx_v.at[pl.ds((c + j) * toks_per_chunk, toks_per_chunk)]
                pltpu.async_copy(table_hbm.at[ids], bufs[j], gsem[j])
            for j in range(_RING):
                ids = idx_v.at[pl.ds((c + j) * toks_per_chunk, toks_per_chunk)]
                pltpu.make_async_copy(table_hbm.at[ids], bufs[j],
                                      gsem[j]).wait()
                b = sent0 + (c + j) * _SPC
                for i in range(_SPC):
                    pltpu.async_copy(bufs[j].at[pl.ds(i * seq, seq)],
                                     out_hbm.at[b + i], wsem[j])
            return carry

        lax.fori_loop(0, n_rounds, round_body, 0)
        for j in range(_RING):
            b = sent0 + (chunks_per_w - _RING + j) * _SPC
            for i in range(_SPC):
                pltpu.make_async_copy(bufs[j].at[pl.ds(i * seq, seq)],
                                      out_hbm.at[b + i], wsem[j]).wait()

    return gather_kernel


def kernel(sentence, word_vectors):
    batch, seq = sentence.shape
    idx = sentence.reshape(batch * seq).astype(jnp.int32)
    return _make_gather(batch, seq)(idx, word_vectors)
